# baseline (speedup 1.0000x reference)
import functools

import jax
import jax.numpy as jnp
from jax import lax
from jax.experimental import pallas as pl
from jax.experimental.pallas import tpu as pltpu
from jax.experimental.pallas import tpu_sc as plsc


def _ln(x, g, b):
    m = jnp.mean(x, axis=-1, keepdims=True)
    v = jnp.mean((x - m) ** 2, axis=-1, keepdims=True)
    return (x - m) / jnp.sqrt(v + 1e-5) * g + b


def _mlp_body(part_meta, *refs):
    it = iter(refs)
    xs = []
    for na, ncnt in part_meta:
        arrs = [next(it) for _ in range(na)]
        cnts = [next(it) for _ in range(ncnt)]
        x = arrs[0][...]
        for a in arrs[1:]:
            x = x + a[...]
        if ncnt:
            ct = cnts[0][...][:, :1]
            for cr in cnts[1:]:
                ct = ct + cr[...][:, :1]
            x = x / jnp.maximum(ct, 1.0)
        xs.append(x)
    w1s = [next(it) for _ in range(len(part_meta))]
    b1, g1, be1, w2, b2, g2, be2 = (next(it) for _ in range(7))
    out_ref = next(it)

    acc = None
    for x, w_ref in zip(xs, w1s):
        pm = jnp.dot(x.astype(jnp.bfloat16), w_ref[...],
                     preferred_element_type=jnp.float32)
        acc = pm if acc is None else acc + pm
    h = jnp.maximum(acc + b1[...], 0.0)
    h = _ln(h, g1[...], be1[...])
    o = jnp.maximum(
        jnp.dot(h.astype(jnp.bfloat16), w2[...],
                preferred_element_type=jnp.float32) + b2[...], 0.0)
    out_ref[...] = _ln(o, g2[...], be2[...])


def _fused_mlp(parts, p, block_rows):
    n = parts[0][0][0].shape[0]
    assert n % block_rows == 0, (n, block_rows)
    dins = [arrs[0].shape[1] for arrs, _ in parts]
    dh = p['W1'].shape[1]
    dout = p['W2'].shape[1]
    part_meta = tuple((len(arrs), len(cnts)) for arrs, cnts in parts)

    offs = [0]
    for d in dins:
        offs.append(offs[-1] + d)
    w1s = [lax.slice(p['W1'], (offs[k], 0), (offs[k + 1], dh))
           .astype(jnp.bfloat16) for k in range(len(parts))]
    w2b = p['W2'].astype(jnp.bfloat16)

    operands = []
    in_specs = []
    for (arrs, cnts), d in zip(parts, dins):
        for a in arrs:
            operands.append(a)
            in_specs.append(pl.BlockSpec((block_rows, d), lambda i: (i, 0)))
        for cr in cnts:
            operands.append(cr)
            in_specs.append(
                pl.BlockSpec((block_rows, cr.shape[1]), lambda i: (i, 0)))
    for w, d in zip(w1s, dins):
        operands.append(w)
        in_specs.append(pl.BlockSpec((d, dh), lambda i: (0, 0)))
    for name, width in (('b1', dh), ('g1', dh), ('be1', dh),
                        ('W2', None), ('b2', dout), ('g2', dout),
                        ('be2', dout)):
        if width is None:
            operands.append(w2b)
            in_specs.append(pl.BlockSpec((dh, dout), lambda i: (0, 0)))
        else:
            operands.append(p[name].reshape(1, width))
            in_specs.append(pl.BlockSpec((1, width), lambda i: (0, 0)))

    return pl.pallas_call(
        functools.partial(_mlp_body, part_meta),
        grid=(n // block_rows,),
        in_specs=in_specs,
        out_specs=pl.BlockSpec((block_rows, dout), lambda i: (i, 0)),
        out_shape=jax.ShapeDtypeStruct((n, dout), jnp.float32),
    )(*operands)


_NC, _NS, _NW = 2, 16, 32
_CH = 128


@functools.cache
def _make_sc_gather(v, d, b):
    nch = b // _CH
    mesh = plsc.VectorSubcoreMesh(core_axis_name="c", subcore_axis_name="s")

    @functools.partial(
        pl.kernel, mesh=mesh,
        out_type=jax.ShapeDtypeStruct((b, d), jnp.float32),
        scratch_types=[
            pltpu.VMEM((2, _CH), jnp.int32),
            pltpu.VMEM((2, _CH, d), jnp.float32),
            pltpu.SemaphoreType.DMA,
            pltpu.SemaphoreType.DMA,
            pltpu.SemaphoreType.DMA,
            pltpu.SemaphoreType.DMA,
            pltpu.SemaphoreType.DMA,
        ],
    )
    def k(table_hbm, idx_hbm, out_hbm, idx_v, rows_v,
          isem0, isem1, gsem, wsem0, wsem1):
        wid = lax.axis_index("s") * _NC + lax.axis_index("c")
        nw = (nch - wid + _NW - 1) // _NW

        def base(i):
            return (wid + i * _NW) * _CH

        def start_idx(i, bi, sem):
            pltpu.async_copy(idx_hbm.at[pl.ds(base(i), _CH)], idx_v.at[bi],
                             sem)

        def wait_idx(i, bi, sem):
            pltpu.make_async_copy(idx_hbm.at[pl.ds(base(i), _CH)],
                                  idx_v.at[bi], sem).wait()

        def wait_write(i, bi, sem):
            pltpu.make_async_copy(rows_v.at[bi],
                                  out_hbm.at[pl.ds(base(i), _CH)],
                                  sem).wait()

        @pl.when(nw > 0)
        def _():
            start_idx(0, 0, isem0)

        npair = (nw + 1) // 2

        def body(jp, carry):
            i0 = 2 * jp
            i1 = i0 + 1
            wait_idx(i0, 0, isem0)

            @pl.when(i1 < nw)
            def _():
                start_idx(i1, 1, isem1)

            @pl.when(jp > 0)
            def _():
                wait_write(i0 - 2, 0, wsem0)

            pltpu.async_copy(table_hbm.at[idx_v.at[0]], rows_v.at[0],
                             gsem).wait()
            pltpu.async_copy(rows_v.at[0], out_hbm.at[pl.ds(base(i0), _CH)],
                             wsem0)

            @pl.when(i1 < nw)
            def _():
                wait_idx(i1, 1, isem1)

                @pl.when(i0 + 2 < nw)
                def _():
                    start_idx(i0 + 2, 0, isem0)

                @pl.when(jp > 0)
                def _():
                    wait_write(i1 - 2, 1, wsem1)

                pltpu.async_copy(table_hbm.at[idx_v.at[1]], rows_v.at[1],
                                 gsem).wait()
                pltpu.async_copy(rows_v.at[1],
                                 out_hbm.at[pl.ds(base(i1), _CH)], wsem1)

            return carry

        lax.fori_loop(0, npair, body, 0, unroll=False)

        @pl.when(nw > 0)
        def _():
            wait_write(2 * ((nw - 1) // 2), 0, wsem0)

        @pl.when(nw > 1)
        def _():
            wait_write(2 * ((nw - 2) // 2) + 1, 1, wsem1)

    return k


def _gather_rows(table, idx):
    return _make_sc_gather(table.shape[0], table.shape[1], idx.shape[0])(
        table, idx)


_NP = 10240
_RPT = _NP // _NS


@functools.cache
def _make_sc_partial(b, col0, use_ones):
    nch = b // _CH
    nch_sc = nch // _NC
    mesh = plsc.VectorSubcoreMesh(core_axis_name="c", subcore_axis_name="s")

    @functools.partial(
        pl.kernel, mesh=mesh,
        out_type=jax.ShapeDtypeStruct((_NC, _NP, _CH), jnp.float32),
        scratch_types=[
            pltpu.VMEM((2, _CH), jnp.int32),
            pltpu.VMEM((2, _CH, _CH), jnp.float32),
            pltpu.VMEM_SHARED((_NP, _CH), jnp.float32),
            pltpu.SemaphoreType.DMA,
            pltpu.SemaphoreType.DMA,
            pltpu.SemaphoreType.DMA,
            pltpu.SemaphoreType.DMA,
        ],
    )
    def k(src_hbm, idx_hbm, zeros_hbm, out_hbm, idx_v, src_v, sh_sum,
          isem0, isem1, ssem0, ssem1):
        c = lax.axis_index("c")
        s = lax.axis_index("s")

        pltpu.sync_copy(zeros_hbm, sh_sum.at[pl.ds(s * _RPT, _RPT)])
        if use_ones:
            pltpu.sync_copy(src_hbm, src_v.at[0])
        plsc.subcore_barrier()

        nw = (nch_sc - s + _NS - 1) // _NS

        def base(i):
            return (c * nch_sc + s + i * _NS) * _CH

        def start(i, bi, isem, ssem):
            pltpu.async_copy(idx_hbm.at[pl.ds(base(i), _CH)], idx_v.at[bi],
                             isem)
            if not use_ones:
                pltpu.async_copy(src_hbm.at[pl.ds(base(i), _CH),
                                            pl.ds(col0, _CH)],
                                 src_v.at[bi], ssem)

        def wait(i, bi, isem, ssem):
            pltpu.make_async_copy(idx_hbm.at[pl.ds(base(i), _CH)],
                                  idx_v.at[bi], isem).wait()
            if not use_ones:
                pltpu.make_async_copy(src_hbm.at[pl.ds(base(i), _CH),
                                                 pl.ds(col0, _CH)],
                                      src_v.at[bi], ssem).wait()

        def stream(bi):
            sb = 0 if use_ones else bi
            pltpu.sync_copy(src_v.at[sb], sh_sum.at[idx_v.at[bi]], add=True)

        @pl.when(nw > 0)
        def _():
            start(0, 0, isem0, ssem0)

        npair = (nw + 1) // 2

        def body(jp, carry):
            i0 = 2 * jp
            i1 = i0 + 1
            wait(i0, 0, isem0, ssem0)

            @pl.when(i1 < nw)
            def _():
                start(i1, 1, isem1, ssem1)

            stream(0)

            @pl.when(i1 < nw)
            def _():
                wait(i1, 1, isem1, ssem1)

                @pl.when(i0 + 2 < nw)
                def _():
                    start(i0 + 2, 0, isem0, ssem0)

                stream(1)

            return carry

        lax.fori_loop(0, npair, body, 0, unroll=False)
        plsc.subcore_barrier()

        pltpu.sync_copy(sh_sum.at[pl.ds(s * _RPT, _RPT)],
                        out_hbm.at[c, pl.ds(s * _RPT, _RPT)])

    return k


def _segment_sum_counts(src, idx, n):
    b = src.shape[0]
    zeros = jnp.zeros((_RPT, _CH), jnp.float32)
    ones = jnp.ones((_CH, _CH), jnp.float32)
    p0 = _make_sc_partial(b, 0, False)(src, idx, zeros)
    p1 = _make_sc_partial(b, _CH, False)(src, idx, zeros)
    pc = _make_sc_partial(b, 0, True)(ones, idx, zeros)
    return ((p0[0][:n], p0[1][:n]), (p1[0][:n], p1[1][:n]),
            (pc[0][:n], pc[1][:n]))


N_INS = 10000
N_LAB = 10000
_RB_EDGE = 1000
_RB_NODE = 1000


def kernel(node_ins, edge_index_ins, edge_attr_ins, node_label,
           edge_index_label, edge_attr_label, edge_index_cross,
           edge_attr_cross, params):
    p = params
    nI = _fused_mlp([([node_ins], [])], p['enc_nI'], _RB_NODE)
    eI = _fused_mlp([([edge_attr_ins], [])], p['enc_eI'], _RB_EDGE)
    nL = _fused_mlp([([node_label], [])], p['enc_nL'], _RB_NODE)
    eL = _fused_mlp([([edge_attr_label], [])], p['enc_eL'], _RB_EDGE)
    eC = _fused_mlp([([edge_attr_cross], [])], p['enc_eC'], _RB_EDGE)

    row_i, col_i = edge_index_ins[0], edge_index_ins[1]
    row_c, col_c = edge_index_cross[0], edge_index_cross[1]
    row_l, col_l = edge_index_label[0], edge_index_label[1]

    def mean_parts(sums, counts):
        (s0, s1, cc) = sums[0], sums[1], counts
        return [(list(s0), list(cc)), (list(s1), list(cc))]

    gI = _gather_rows(nI, row_i)
    m_inner = _fused_mlp([([gI], []), ([eI], [])], p['mlp_ins_inner'],
                         _RB_EDGE)
    s_in0, s_in1, c_in = _segment_sum_counts(m_inner, col_i, N_INS)

    gLc = _gather_rows(nL, col_c)
    m_inter = _fused_mlp([([gLc], []), ([eC], [])], p['mlp_ins_inter'],
                         _RB_EDGE)
    s_it0, s_it1, c_it = _segment_sum_counts(m_inter, row_c, N_INS)

    nI_new = _fused_mlp(
        [([nI], [])] + mean_parts((s_in0, s_in1), c_in)
        + mean_parts((s_it0, s_it1), c_it), p['mlp_ins'], _RB_NODE)

    gL = _gather_rows(nL, row_l)
    l_inner = _fused_mlp([([gL], []), ([eL], [])], p['mlp_lab_inner'],
                         _RB_EDGE)
    t_in0, t_in1, d_in = _segment_sum_counts(l_inner, col_l, N_LAB)

    gIc = _gather_rows(nI_new, row_c)
    l_inter = _fused_mlp([([gIc], []), ([eC], [])], p['mlp_lab_inter'],
                         _RB_EDGE)
    t_it0, t_it1, d_it = _segment_sum_counts(l_inter, col_c, N_LAB)

    nL_new = _fused_mlp(
        [([nL], [])] + mean_parts((t_in0, t_in1), d_in)
        + mean_parts((t_it0, t_it1), d_it), p['mlp_lab'], _RB_NODE)

    return ((nI_new, edge_index_ins, eI), (nL_new, edge_index_label, eL),
            (edge_index_cross, eC))


# speedup vs baseline: 1.1371x; 1.1371x over previous
import functools

import jax
import jax.numpy as jnp
from jax import lax
from jax.experimental import pallas as pl
from jax.experimental.pallas import tpu as pltpu
from jax.experimental.pallas import tpu_sc as plsc


def _ln(x, g, b):
    m = jnp.mean(x, axis=-1, keepdims=True)
    v = jnp.mean((x - m) ** 2, axis=-1, keepdims=True)
    return (x - m) / jnp.sqrt(v + 1e-5) * g + b


def _mlp_body(part_meta, *refs):
    it = iter(refs)
    xs = []
    for na, ncnt in part_meta:
        arrs = [next(it) for _ in range(na)]
        cnts = [next(it) for _ in range(ncnt)]
        x = arrs[0][...]
        for a in arrs[1:]:
            x = x + a[...]
        if ncnt:
            ct = cnts[0][...][:, :1]
            for cr in cnts[1:]:
                ct = ct + cr[...][:, :1]
            x = x / jnp.maximum(ct, 1.0)
        xs.append(x)
    w1s = [next(it) for _ in range(len(part_meta))]
    b1, w2g, vrow, urow, g2, be2 = (next(it) for _ in range(6))
    out_ref = next(it)

    acc = None
    for x, w_ref in zip(xs, w1s):
        pm = jnp.dot(x, w_ref[...], preferred_element_type=jnp.float32)
        acc = pm if acc is None else acc + pm
    h = jnp.maximum(acc + b1[...], 0.0)
    dh = h.shape[1]
    m = jnp.sum(h, axis=-1, keepdims=True) * (1.0 / dh)
    var = jnp.sum(h * h, axis=-1, keepdims=True) * (1.0 / dh) - m * m
    inv = lax.rsqrt(var + 1e-5)
    a = jnp.dot(h, w2g[...], preferred_element_type=jnp.float32)
    o = jnp.maximum(inv * a - (inv * m) * vrow[...] + urow[...], 0.0)
    dout = o.shape[1]
    mo = jnp.sum(o, axis=-1, keepdims=True) * (1.0 / dout)
    varo = jnp.sum(o * o, axis=-1, keepdims=True) * (1.0 / dout) - mo * mo
    invo = lax.rsqrt(varo + 1e-5)
    out_ref[...] = (o - mo) * (invo * g2[...]) + be2[...]


def _fused_mlp(parts, p, block_rows):
    n = parts[0][0][0].shape[0]
    assert n % block_rows == 0, (n, block_rows)
    dins = [arrs[0].shape[1] for arrs, _ in parts]
    dh = p['W1'].shape[1]
    dout = p['W2'].shape[1]
    part_meta = tuple((len(arrs), len(cnts)) for arrs, cnts in parts)

    offs = [0]
    for d in dins:
        offs.append(offs[-1] + d)
    w1s = [lax.slice(p['W1'], (offs[k], 0), (offs[k + 1], dh))
           for k in range(len(parts))]
    w2g = p['g1'][:, None] * p['W2']
    vrow = (p['g1'] @ p['W2']).reshape(1, dout)
    urow = (p['be1'] @ p['W2'] + p['b2']).reshape(1, dout)

    operands = []
    in_specs = []
    for (arrs, cnts), d in zip(parts, dins):
        for a in arrs:
            operands.append(a)
            in_specs.append(pl.BlockSpec((block_rows, d), lambda i: (i, 0)))
        for cr in cnts:
            operands.append(cr)
            in_specs.append(
                pl.BlockSpec((block_rows, cr.shape[1]), lambda i: (i, 0)))
    for w, d in zip(w1s, dins):
        operands.append(w)
        in_specs.append(pl.BlockSpec((d, dh), lambda i: (0, 0)))
    for arr in (p['b1'].reshape(1, dh), w2g, vrow, urow,
                p['g2'].reshape(1, dout), p['be2'].reshape(1, dout)):
        operands.append(arr)
        in_specs.append(
            pl.BlockSpec(arr.shape, lambda i, nd=arr.ndim: (0,) * nd))

    return pl.pallas_call(
        functools.partial(_mlp_body, part_meta),
        grid=(n // block_rows,),
        in_specs=in_specs,
        out_specs=pl.BlockSpec((block_rows, dout), lambda i: (i, 0)),
        out_shape=jax.ShapeDtypeStruct((n, dout), jnp.float32),
    )(*operands)


_NC, _NS, _NW = 2, 16, 32
_CH = 128


@functools.cache
def _make_sc_gather(v, d, b):
    nch = b // _CH
    mesh = plsc.VectorSubcoreMesh(core_axis_name="c", subcore_axis_name="s")

    @functools.partial(
        pl.kernel, mesh=mesh,
        out_type=jax.ShapeDtypeStruct((b, d), jnp.float32),
        scratch_types=[
            pltpu.VMEM((2, _CH), jnp.int32),
            pltpu.VMEM((2, _CH, d), jnp.float32),
            pltpu.SemaphoreType.DMA,
            pltpu.SemaphoreType.DMA,
            pltpu.SemaphoreType.DMA,
            pltpu.SemaphoreType.DMA,
            pltpu.SemaphoreType.DMA,
        ],
    )
    def k(table_hbm, idx_hbm, out_hbm, idx_v, rows_v,
          isem0, isem1, gsem, wsem0, wsem1):
        wid = lax.axis_index("s") * _NC + lax.axis_index("c")
        nw = (nch - wid + _NW - 1) // _NW

        def base(i):
            return (wid + i * _NW) * _CH

        def start_idx(i, bi, sem):
            pltpu.async_copy(idx_hbm.at[pl.ds(base(i), _CH)], idx_v.at[bi],
                             sem)

        def wait_idx(i, bi, sem):
            pltpu.make_async_copy(idx_hbm.at[pl.ds(base(i), _CH)],
                                  idx_v.at[bi], sem).wait()

        def wait_write(i, bi, sem):
            pltpu.make_async_copy(rows_v.at[bi],
                                  out_hbm.at[pl.ds(base(i), _CH)],
                                  sem).wait()

        @pl.when(nw > 0)
        def _():
            start_idx(0, 0, isem0)

        npair = (nw + 1) // 2

        def body(jp, carry):
            i0 = 2 * jp
            i1 = i0 + 1
            wait_idx(i0, 0, isem0)

            @pl.when(i1 < nw)
            def _():
                start_idx(i1, 1, isem1)

            @pl.when(jp > 0)
            def _():
                wait_write(i0 - 2, 0, wsem0)

            pltpu.async_copy(table_hbm.at[idx_v.at[0]], rows_v.at[0],
                             gsem).wait()
            pltpu.async_copy(rows_v.at[0], out_hbm.at[pl.ds(base(i0), _CH)],
                             wsem0)

            @pl.when(i1 < nw)
            def _():
                wait_idx(i1, 1, isem1)

                @pl.when(i0 + 2 < nw)
                def _():
                    start_idx(i0 + 2, 0, isem0)

                @pl.when(jp > 0)
                def _():
                    wait_write(i1 - 2, 1, wsem1)

                pltpu.async_copy(table_hbm.at[idx_v.at[1]], rows_v.at[1],
                                 gsem).wait()
                pltpu.async_copy(rows_v.at[1],
                                 out_hbm.at[pl.ds(base(i1), _CH)], wsem1)

            return carry

        lax.fori_loop(0, npair, body, 0, unroll=False)

        @pl.when(nw > 0)
        def _():
            wait_write(2 * ((nw - 1) // 2), 0, wsem0)

        @pl.when(nw > 1)
        def _():
            wait_write(2 * ((nw - 2) // 2) + 1, 1, wsem1)

    return k


def _gather_rows(table, idx):
    return _make_sc_gather(table.shape[0], table.shape[1], idx.shape[0])(
        table, idx)


_NP = 10240
_RPT = _NP // _NS


@functools.cache
def _make_sc_partial(b, col0, use_ones):
    nch = b // _CH
    nch_sc = nch // _NC
    mesh = plsc.VectorSubcoreMesh(core_axis_name="c", subcore_axis_name="s")

    @functools.partial(
        pl.kernel, mesh=mesh,
        out_type=jax.ShapeDtypeStruct((_NC, _NP, _CH), jnp.float32),
        scratch_types=[
            pltpu.VMEM((2, _CH), jnp.int32),
            pltpu.VMEM((2, _CH, _CH), jnp.float32),
            pltpu.VMEM_SHARED((_NP, _CH), jnp.float32),
            pltpu.SemaphoreType.DMA,
            pltpu.SemaphoreType.DMA,
            pltpu.SemaphoreType.DMA,
            pltpu.SemaphoreType.DMA,
        ],
    )
    def k(src_hbm, idx_hbm, zeros_hbm, out_hbm, idx_v, src_v, sh_sum,
          isem0, isem1, ssem0, ssem1):
        c = lax.axis_index("c")
        s = lax.axis_index("s")

        pltpu.sync_copy(zeros_hbm, sh_sum.at[pl.ds(s * _RPT, _RPT)])
        if use_ones:
            pltpu.sync_copy(src_hbm, src_v.at[0])
        plsc.subcore_barrier()

        nw = (nch_sc - s + _NS - 1) // _NS

        def base(i):
            return (c * nch_sc + s + i * _NS) * _CH

        def start(i, bi, isem, ssem):
            pltpu.async_copy(idx_hbm.at[pl.ds(base(i), _CH)], idx_v.at[bi],
                             isem)
            if not use_ones:
                pltpu.async_copy(src_hbm.at[pl.ds(base(i), _CH),
                                            pl.ds(col0, _CH)],
                                 src_v.at[bi], ssem)

        def wait(i, bi, isem, ssem):
            pltpu.make_async_copy(idx_hbm.at[pl.ds(base(i), _CH)],
                                  idx_v.at[bi], isem).wait()
            if not use_ones:
                pltpu.make_async_copy(src_hbm.at[pl.ds(base(i), _CH),
                                                 pl.ds(col0, _CH)],
                                      src_v.at[bi], ssem).wait()

        def stream(bi):
            sb = 0 if use_ones else bi
            pltpu.sync_copy(src_v.at[sb], sh_sum.at[idx_v.at[bi]], add=True)

        @pl.when(nw > 0)
        def _():
            start(0, 0, isem0, ssem0)

        npair = (nw + 1) // 2

        def body(jp, carry):
            i0 = 2 * jp
            i1 = i0 + 1
            wait(i0, 0, isem0, ssem0)

            @pl.when(i1 < nw)
            def _():
                start(i1, 1, isem1, ssem1)

            stream(0)

            @pl.when(i1 < nw)
            def _():
                wait(i1, 1, isem1, ssem1)

                @pl.when(i0 + 2 < nw)
                def _():
                    start(i0 + 2, 0, isem0, ssem0)

                stream(1)

            return carry

        lax.fori_loop(0, npair, body, 0, unroll=False)
        plsc.subcore_barrier()

        pltpu.sync_copy(sh_sum.at[pl.ds(s * _RPT, _RPT)],
                        out_hbm.at[c, pl.ds(s * _RPT, _RPT)])

    return k


def _segment_sum_counts(src, idx, n):
    b = src.shape[0]
    zeros = jnp.zeros((_RPT, _CH), jnp.float32)
    ones = jnp.ones((_CH, _CH), jnp.float32)
    p0 = _make_sc_partial(b, 0, False)(src, idx, zeros)
    p1 = _make_sc_partial(b, _CH, False)(src, idx, zeros)
    pc = _make_sc_partial(b, 0, True)(ones, idx, zeros)
    return ((p0[0][:n], p0[1][:n]), (p1[0][:n], p1[1][:n]),
            (pc[0][:n], pc[1][:n]))


N_INS = 10000
N_LAB = 10000
_RB_EDGE = 1000
_RB_NODE = 1000


def kernel(node_ins, edge_index_ins, edge_attr_ins, node_label,
           edge_index_label, edge_attr_label, edge_index_cross,
           edge_attr_cross, params):
    p = params
    nI = _fused_mlp([([node_ins], [])], p['enc_nI'], _RB_NODE)
    eI = _fused_mlp([([edge_attr_ins], [])], p['enc_eI'], _RB_EDGE)
    nL = _fused_mlp([([node_label], [])], p['enc_nL'], _RB_NODE)
    eL = _fused_mlp([([edge_attr_label], [])], p['enc_eL'], _RB_EDGE)
    eC = _fused_mlp([([edge_attr_cross], [])], p['enc_eC'], _RB_EDGE)

    row_i, col_i = edge_index_ins[0], edge_index_ins[1]
    row_c, col_c = edge_index_cross[0], edge_index_cross[1]
    row_l, col_l = edge_index_label[0], edge_index_label[1]

    def mean_parts(sums, counts):
        (s0, s1, cc) = sums[0], sums[1], counts
        return [(list(s0), list(cc)), (list(s1), list(cc))]

    gI = _gather_rows(nI, row_i)
    m_inner = _fused_mlp([([gI], []), ([eI], [])], p['mlp_ins_inner'],
                         _RB_EDGE)
    s_in0, s_in1, c_in = _segment_sum_counts(m_inner, col_i, N_INS)

    gLc = _gather_rows(nL, col_c)
    m_inter = _fused_mlp([([gLc], []), ([eC], [])], p['mlp_ins_inter'],
                         _RB_EDGE)
    s_it0, s_it1, c_it = _segment_sum_counts(m_inter, row_c, N_INS)

    nI_new = _fused_mlp(
        [([nI], [])] + mean_parts((s_in0, s_in1), c_in)
        + mean_parts((s_it0, s_it1), c_it), p['mlp_ins'], _RB_NODE)

    gL = _gather_rows(nL, row_l)
    l_inner = _fused_mlp([([gL], []), ([eL], [])], p['mlp_lab_inner'],
                         _RB_EDGE)
    t_in0, t_in1, d_in = _segment_sum_counts(l_inner, col_l, N_LAB)

    gIc = _gather_rows(nI_new, row_c)
    l_inter = _fused_mlp([([gIc], []), ([eC], [])], p['mlp_lab_inter'],
                         _RB_EDGE)
    t_it0, t_it1, d_it = _segment_sum_counts(l_inter, col_c, N_LAB)

    nL_new = _fused_mlp(
        [([nL], [])] + mean_parts((t_in0, t_in1), d_in)
        + mean_parts((t_it0, t_it1), d_it), p['mlp_lab'], _RB_NODE)

    return ((nI_new, edge_index_ins, eI), (nL_new, edge_index_label, eL),
            (edge_index_cross, eC))
